# Initial kernel scaffold; baseline (speedup 1.0000x reference)
import jax
import jax.numpy as jnp
from jax.experimental import pallas as pl


def kernel(x, Wq, Wkv, Wout, b_out):
    raise NotImplementedError("write your pallas kernel here")


# speedup vs baseline: 1.9365x; 1.9365x over previous
import jax
import jax.numpy as jnp
from jax.experimental import pallas as pl

_B, _N, _DIM = 2, 2048, 1024
_H, _DH = 16, 64
_INNER = _H * _DH
_SCALE = _DH ** -0.5

_BLK = 256
_NI = _N // _BLK
_KVBLK = 512
_NKV = _N // _KVBLK


def _kv_kernel(x_ref, wkv_ref, k_ref, v_ref):
    x = x_ref[0]
    kv = jax.lax.dot_general(x, wkv_ref[...], (((1,), (1,)), ((), ())),
                             preferred_element_type=jnp.float32)
    k_ref[0] = kv[:, :_DH]
    v_ref[0] = kv[:, _DH:]


def _attn_kernel(x_ref, wq_ref, k_ref, v_ref, wout_ref, bout_ref, out_ref):
    i = pl.program_id(1)
    h = pl.program_id(2)
    x = x_ref[0]
    q = jax.lax.dot_general(x, wq_ref[...], (((1,), (1,)), ((), ())),
                            preferred_element_type=jnp.float32) * _SCALE
    k = k_ref[0]
    sim = jax.lax.dot_general(q, k, (((1,), (1,)), ((), ())),
                              preferred_element_type=jnp.float32)
    row = i * _BLK + jax.lax.broadcasted_iota(jnp.int32, (_BLK, _N), 0)
    col = jax.lax.broadcasted_iota(jnp.int32, (_BLK, _N), 1)
    sim = jnp.where(col > row, -jnp.inf, sim)
    m = jnp.max(sim, axis=-1, keepdims=True)
    e = jnp.exp(sim - m)
    attn = e / jnp.sum(e, axis=-1, keepdims=True)
    lv = jax.lax.dot_general(attn, v_ref[0], (((1,), (0,)), ((), ())),
                             preferred_element_type=jnp.float32)
    contrib = jax.lax.dot_general(lv, wout_ref[0], (((1,), (1,)), ((), ())),
                                  preferred_element_type=jnp.float32)

    @pl.when(h == 0)
    def _init():
        out_ref[0] = contrib + bout_ref[...]

    @pl.when(h != 0)
    def _acc():
        out_ref[0] += contrib


def kernel(x, Wq, Wkv, Wout, b_out):
    k, v = pl.pallas_call(
        _kv_kernel,
        grid=(_B, _NKV),
        in_specs=[
            pl.BlockSpec((1, _KVBLK, _DIM), lambda b, i: (b, i, 0)),
            pl.BlockSpec((2 * _DH, _DIM), lambda b, i: (0, 0)),
        ],
        out_specs=[
            pl.BlockSpec((1, _KVBLK, _DH), lambda b, i: (b, i, 0)),
            pl.BlockSpec((1, _KVBLK, _DH), lambda b, i: (b, i, 0)),
        ],
        out_shape=[
            jax.ShapeDtypeStruct((_B, _N, _DH), jnp.float32),
            jax.ShapeDtypeStruct((_B, _N, _DH), jnp.float32),
        ],
    )(x, Wkv)

    out = pl.pallas_call(
        _attn_kernel,
        grid=(_B, _NI, _H),
        in_specs=[
            pl.BlockSpec((1, _BLK, _DIM), lambda b, i, h: (b, i, 0)),
            pl.BlockSpec((_DH, _DIM), lambda b, i, h: (h, 0)),
            pl.BlockSpec((1, _N, _DH), lambda b, i, h: (b, 0, 0)),
            pl.BlockSpec((1, _N, _DH), lambda b, i, h: (b, 0, 0)),
            pl.BlockSpec((1, _DIM, _DH), lambda b, i, h: (h, 0, 0)),
            pl.BlockSpec((1, _DIM), lambda b, i, h: (0, 0)),
        ],
        out_specs=pl.BlockSpec((1, _BLK, _DIM), lambda b, i, h: (b, i, 0)),
        out_shape=jax.ShapeDtypeStruct((_B, _N, _DIM), jnp.float32),
    )(x, Wq, k, v, Wout.reshape(_DIM, _H, _DH).transpose(1, 0, 2),
      b_out.reshape(1, _DIM))
    return out


# speedup vs baseline: 2.0834x; 1.0759x over previous
import jax
import jax.numpy as jnp
from jax.experimental import pallas as pl

_B, _N, _DIM = 2, 2048, 1024
_H, _DH = 16, 64
_INNER = _H * _DH
_SCALE = _DH ** -0.5

_BLK = 256
_NI = _N // _BLK
_KVBLK = 512
_NKV = _N // _KVBLK


def _dot(a, b, dims):
    return jax.lax.dot_general(a, b, (dims, ((), ())),
                               preferred_element_type=jnp.float32)


def _kv_kernel(x_ref, wkv_ref, k_ref, v_ref):
    kv = _dot(x_ref[0], wkv_ref[...], ((1,), (1,)))
    kv = kv.astype(jnp.bfloat16)
    k_ref[0] = kv[:, :_DH]
    v_ref[0] = kv[:, _DH:]


def _attn_kernel(x_ref, wq_ref, k_ref, v_ref, wout_ref, bout_ref, out_ref):
    i = pl.program_id(1)
    h = pl.program_id(2)
    x = x_ref[0]
    q = _dot(x, wq_ref[...], ((1,), (1,))) * _SCALE
    sim = _dot(q.astype(jnp.bfloat16), k_ref[0], ((1,), (1,)))
    row = i * _BLK + jax.lax.broadcasted_iota(jnp.int32, (_BLK, _N), 0)
    col = jax.lax.broadcasted_iota(jnp.int32, (_BLK, _N), 1)
    sim = jnp.where(col > row, -jnp.inf, sim)
    m = jnp.max(sim, axis=-1, keepdims=True)
    e = jnp.exp(sim - m)
    attn = (e / jnp.sum(e, axis=-1, keepdims=True)).astype(jnp.bfloat16)
    lv = _dot(attn, v_ref[0], ((1,), (0,)))
    contrib = _dot(lv.astype(jnp.bfloat16), wout_ref[0], ((1,), (1,)))

    @pl.when(h == 0)
    def _init():
        out_ref[0] = contrib + bout_ref[...]

    @pl.when(h != 0)
    def _acc():
        out_ref[0] += contrib


def kernel(x, Wq, Wkv, Wout, b_out):
    xh = x.astype(jnp.bfloat16)
    k, v = pl.pallas_call(
        _kv_kernel,
        grid=(_B, _NKV),
        in_specs=[
            pl.BlockSpec((1, _KVBLK, _DIM), lambda b, i: (b, i, 0)),
            pl.BlockSpec((2 * _DH, _DIM), lambda b, i: (0, 0)),
        ],
        out_specs=[
            pl.BlockSpec((1, _KVBLK, _DH), lambda b, i: (b, i, 0)),
            pl.BlockSpec((1, _KVBLK, _DH), lambda b, i: (b, i, 0)),
        ],
        out_shape=[
            jax.ShapeDtypeStruct((_B, _N, _DH), jnp.bfloat16),
            jax.ShapeDtypeStruct((_B, _N, _DH), jnp.bfloat16),
        ],
    )(xh, Wkv.astype(jnp.bfloat16))

    out = pl.pallas_call(
        _attn_kernel,
        grid=(_B, _NI, _H),
        in_specs=[
            pl.BlockSpec((1, _BLK, _DIM), lambda b, i, h: (b, i, 0)),
            pl.BlockSpec((_DH, _DIM), lambda b, i, h: (h, 0)),
            pl.BlockSpec((1, _N, _DH), lambda b, i, h: (b, 0, 0)),
            pl.BlockSpec((1, _N, _DH), lambda b, i, h: (b, 0, 0)),
            pl.BlockSpec((1, _DIM, _DH), lambda b, i, h: (h, 0, 0)),
            pl.BlockSpec((1, _DIM), lambda b, i, h: (0, 0)),
        ],
        out_specs=pl.BlockSpec((1, _BLK, _DIM), lambda b, i, h: (b, i, 0)),
        out_shape=jax.ShapeDtypeStruct((_B, _N, _DIM), jnp.float32),
    )(xh, Wq.astype(jnp.bfloat16), k, v,
      Wout.reshape(_DIM, _H, _DH).transpose(1, 0, 2).astype(jnp.bfloat16),
      b_out.reshape(1, _DIM))
    return out
